# baseline (speedup 1.0000x reference)
import functools

import jax
import jax.numpy as jnp
from jax import lax
from jax.experimental import pallas as pl
from jax.experimental.pallas import tpu as pltpu
from jax.experimental.pallas import tpu_sc as plsc

N = 10000
E = 320000
D = 128
H = 64

NC = 2
NS = 16
NW = NC * NS
EPW = E // NW
CHUNK = 80
NCHUNKS = EPW // CHUNK
NP = 10240
RPT = NP // NS
DW = 8


def _sc_mesh():
    return plsc.VectorSubcoreMesh(core_axis_name="c", subcore_axis_name="s")


NBUF = 5


def _make_edge_scatter(F):
    @functools.partial(
        pl.kernel,
        out_type=jax.ShapeDtypeStruct((NC * NP, F), jnp.float32),
        mesh=_sc_mesh(),
        compiler_params=pltpu.CompilerParams(use_tc_tiling_on_sc=False),
        scratch_types=[
            pltpu.VMEM((NCHUNKS, CHUNK), jnp.int32),
            pltpu.VMEM((NCHUNKS, CHUNK), jnp.int32),
            pltpu.VMEM((RPT, F), jnp.float32),
            pltpu.VMEM_SHARED((NP, F), jnp.float32),
        ] + [pltpu.VMEM((CHUNK, F), jnp.float32)] * NBUF
          + [pltpu.SemaphoreType.DMA] * NBUF,
    )
    def edge_scatter(y, src3, dst3, zeros, out, src_all, dst_all, bounce_v,
                     acc, *bufs_sems):
        bufs = bufs_sems[:NBUF]
        sems = bufs_sems[NBUF:]
        c = lax.axis_index("c")
        s = lax.axis_index("s")
        row0 = s * RPT
        wid = c * NS + s
        for i in range(RPT // CHUNK):
            pltpu.sync_copy(zeros, acc.at[pl.ds(row0 + i * CHUNK, CHUNK)])

        pltpu.sync_copy(src3.at[wid], src_all)
        pltpu.sync_copy(dst3.at[wid], dst_all)
        plsc.subcore_barrier()

        for b in range(NBUF):
            pltpu.async_copy(y.at[src_all.at[b]], bufs[b], sems[b])

        def outer(j, carry):
            k0 = j * NBUF
            for b in range(NBUF):
                k = k0 + b
                pltpu.make_async_copy(y.at[src_all.at[0]], bufs[b],
                                      sems[b]).wait()
                pltpu.sync_copy(bufs[b], acc.at[dst_all.at[k]], add=True)
                nk = k + NBUF

                @pl.when(nk < NCHUNKS)
                def _():
                    pltpu.async_copy(y.at[src_all.at[nk]], bufs[b],
                                     sems[b])

            return carry

        lax.fori_loop(0, NCHUNKS // NBUF, outer, 0)
        plsc.subcore_barrier()
        pltpu.sync_copy(acc.at[pl.ds(row0, RPT)], bounce_v)
        pltpu.sync_copy(bounce_v, out.at[pl.ds(c * NP + row0, RPT)])

    return edge_scatter


@functools.partial(
    pl.kernel,
    out_type=jax.ShapeDtypeStruct((NC * NP, DW), jnp.float32),
    mesh=_sc_mesh(),
    compiler_params=pltpu.CompilerParams(use_tc_tiling_on_sc=False),
    scratch_types=[
        pltpu.VMEM((NCHUNKS, CHUNK), jnp.int32),
        pltpu.VMEM((CHUNK, DW), jnp.float32),
        pltpu.VMEM((RPT, DW), jnp.float32),
        pltpu.VMEM_SHARED((NP, DW), jnp.float32),
        pltpu.SemaphoreType.DMA,
    ],
)
def _deg_pass(dst3, ones, zeros, out, dst_all, ones_v, bounce_v, acc, dsem):
    c = lax.axis_index("c")
    s = lax.axis_index("s")
    row0 = s * RPT
    for i in range(RPT // CHUNK):
        pltpu.sync_copy(zeros, acc.at[pl.ds(row0 + i * CHUNK, CHUNK)])
    pltpu.sync_copy(ones, ones_v)
    wid = c * NS + s
    pltpu.sync_copy(dst3.at[wid], dst_all)
    plsc.subcore_barrier()

    def body(k, carry):
        pltpu.async_copy(ones_v, acc.at[dst_all.at[k]], dsem, add=True)
        return carry

    lax.fori_loop(0, NCHUNKS, body, 0)

    def drain(k, carry):
        pltpu.make_async_copy(ones_v, acc.at[dst_all.at[0]], dsem).wait()
        return carry

    lax.fori_loop(0, NCHUNKS, drain, 0)
    plsc.subcore_barrier()
    pltpu.sync_copy(acc.at[pl.ds(row0, RPT)], bounce_v)
    pltpu.sync_copy(bounce_v, out.at[pl.ds(c * NP + row0, RPT)])


_scat_h = _make_edge_scatter(H)

_BLK = 1000


def _row_spec(f):
    return pl.BlockSpec((_BLK, f), lambda i: (i, 0))


def _full_spec(r, f):
    return pl.BlockSpec((r, f), lambda i: (0, 0))


def _mm0_body(x_ref, w_ref, d0_ref, d1_ref, y_ref, dis_ref):
    deg = d0_ref[...] + d1_ref[...] + 1.0
    dis = lax.rsqrt(deg)
    dis_ref[...] = dis
    y_ref[...] = jnp.dot(x_ref[...], w_ref[...],
                         preferred_element_type=jnp.float32) * dis


_mm0 = pl.pallas_call(
    _mm0_body,
    grid=(N // _BLK,),
    in_specs=[
        _row_spec(D),
        _full_spec(D, H),
        _row_spec(1),
        _row_spec(1),
    ],
    out_specs=[_row_spec(H), _row_spec(1)],
    out_shape=[
        jax.ShapeDtypeStruct((N, H), jnp.float32),
        jax.ShapeDtypeStruct((N, 1), jnp.float32),
    ],
)


def _make_cmb(fin, fout, relu):
    def body(z0_ref, z1_ref, yp_ref, dis_ref, b_ref, w_ref, yo_ref):
        h = (z0_ref[...] + z1_ref[...] + yp_ref[...]) * dis_ref[...] + b_ref[...]
        if relu:
            h = jnp.maximum(h, 0.0)
        yo_ref[...] = jnp.dot(h, w_ref[...],
                              preferred_element_type=jnp.float32) * dis_ref[...]

    return pl.pallas_call(
        body,
        grid=(N // _BLK,),
        in_specs=[
            _row_spec(fin),
            _row_spec(fin),
            _row_spec(fin),
            _row_spec(1),
            _full_spec(1, fin),
            _full_spec(fin, fout),
        ],
        out_specs=_row_spec(fout),
        out_shape=jax.ShapeDtypeStruct((N, fout), jnp.float32),
    )


_cmb_enc = _make_cmb(H, H, True)


def _mid_body(z0_ref, z1_ref, yp_ref, dis_ref, b_ref, hp_ref):
    h = (z0_ref[...] + z1_ref[...] + yp_ref[...]) * dis_ref[...] + b_ref[...]
    hp_ref[...] = h * dis_ref[...]


_cmb_mid = pl.pallas_call(
    _mid_body,
    grid=(N // _BLK,),
    in_specs=[
        _row_spec(H),
        _row_spec(H),
        _row_spec(H),
        _row_spec(1),
        _full_spec(1, H),
    ],
    out_specs=_row_spec(H),
    out_shape=jax.ShapeDtypeStruct((N, H), jnp.float32),
)


def _dec_body(zh0_ref, zh1_ref, hp_ref, dis_ref, b2_ref, b4_ref, w24_ref,
              ap_ref, hs_ref):
    u = zh0_ref[...] + zh1_ref[...] + hp_ref[...]
    v = jnp.dot(u, w24_ref[...], preferred_element_type=jnp.float32)
    a = jnp.maximum(v[:, :H] * dis_ref[...] + b2_ref[...], 0.0)
    ap_ref[...] = a * dis_ref[...]
    hs_ref[...] = (v[:, H:] * dis_ref[...] + b4_ref[...]).astype(jnp.bfloat16)


_dec = pl.pallas_call(
    _dec_body,
    grid=(N // _BLK,),
    in_specs=[
        _row_spec(H),
        _row_spec(H),
        _row_spec(H),
        _row_spec(1),
        _full_spec(1, H),
        _full_spec(1, H),
        _full_spec(H, 2 * H),
    ],
    out_specs=[_row_spec(H), _row_spec(H)],
    out_shape=[
        jax.ShapeDtypeStruct((N, H), jnp.float32),
        jax.ShapeDtypeStruct((N, H), jnp.bfloat16),
    ],
)


def _fin_body(za0_ref, za1_ref, ap_ref, dis_ref, b3_ref, w3_ref, x_ref):
    u = za0_ref[...] + za1_ref[...] + ap_ref[...]
    x_ref[...] = (jnp.dot(u, w3_ref[...], preferred_element_type=jnp.float32)
                  * dis_ref[...] + b3_ref[...])


_fin = pl.pallas_call(
    _fin_body,
    grid=(N // _BLK,),
    in_specs=[
        _row_spec(H),
        _row_spec(H),
        _row_spec(H),
        _row_spec(1),
        _full_spec(1, D),
        _full_spec(H, D),
    ],
    out_specs=_row_spec(D),
    out_shape=jax.ShapeDtypeStruct((N, D), jnp.float32),
)

_BM, _BN = 1024, 512


def _gram_body(a_ref, b_ref, o_ref):
    o_ref[...] = lax.dot_general(a_ref[...], b_ref[...],
                                 (((1,), (1,)), ((), ())),
                                 preferred_element_type=jnp.float32)


_gram = pl.pallas_call(
    _gram_body,
    grid=(pl.cdiv(N, _BM), pl.cdiv(N, _BN)),
    in_specs=[
        pl.BlockSpec((_BM, H), lambda i, j: (i, 0)),
        pl.BlockSpec((_BN, H), lambda i, j: (j, 0)),
    ],
    out_specs=pl.BlockSpec((_BM, _BN), lambda i, j: (i, j)),
    out_shape=jax.ShapeDtypeStruct((N, N), jnp.float32),
)


def kernel(x, edge_index, W0, b0, W1, b1, W2, b2, W3, b3, W4, b4):
    src3 = edge_index[0].reshape(NW, NCHUNKS, CHUNK)
    dst3 = edge_index[1].reshape(NW, NCHUNKS, CHUNK)
    zeros_h = jnp.zeros((CHUNK, H), jnp.float32)
    zeros_w = jnp.zeros((CHUNK, DW), jnp.float32)
    ones_w = jnp.ones((CHUNK, DW), jnp.float32)

    degp = _deg_pass(dst3, ones_w, zeros_w)
    y0, dis = _mm0(x, W0, degp[:N, :1], degp[NP:NP + N, :1])

    z0 = _scat_h(y0, src3, dst3, zeros_h)
    y1 = _cmb_enc(z0[:N], z0[NP:NP + N], y0, dis, b0[None, :], W1)

    z1 = _scat_h(y1, src3, dst3, zeros_h)
    hp = _cmb_mid(z1[:N], z1[NP:NP + N], y1, dis, b1[None, :])

    zh = _scat_h(hp, src3, dst3, zeros_h)
    W24 = jnp.concatenate([W2, W4], axis=1)
    ap, hs = _dec(zh[:N], zh[NP:NP + N], hp, dis, b2[None, :], b4[None, :],
                  W24)

    za = _scat_h(ap, src3, dst3, zeros_h)
    x_ = _fin(za[:N], za[NP:NP + N], ap, dis, b3[None, :], W3)

    s_ = _gram(hs, hs)
    return (x_, s_)


# speedup vs baseline: 1.0543x; 1.0543x over previous
import functools

import jax
import jax.numpy as jnp
from jax import lax
from jax.experimental import pallas as pl
from jax.experimental.pallas import tpu as pltpu
from jax.experimental.pallas import tpu_sc as plsc

N = 10000
E = 320000
D = 128
H = 64

NC = 2
NS = 16
NW = NC * NS
EPW = E // NW
CHUNK = 80
NCHUNKS = EPW // CHUNK
NP = 10240
RPT = NP // NS
DW = 8


def _sc_mesh():
    return plsc.VectorSubcoreMesh(core_axis_name="c", subcore_axis_name="s")


NBUF = 5


def _make_edge_scatter(F):
    @functools.partial(
        pl.kernel,
        out_type=jax.ShapeDtypeStruct((NC * NP, F), jnp.float32),
        mesh=_sc_mesh(),
        compiler_params=pltpu.CompilerParams(use_tc_tiling_on_sc=False),
        scratch_types=[
            pltpu.VMEM((NCHUNKS, CHUNK), jnp.int32),
            pltpu.VMEM((NCHUNKS, CHUNK), jnp.int32),
            pltpu.VMEM((RPT, F), jnp.float32),
            pltpu.VMEM_SHARED((NP, F), jnp.float32),
        ] + [pltpu.VMEM((CHUNK, F), jnp.float32)] * NBUF
          + [pltpu.SemaphoreType.DMA] * NBUF,
    )
    def edge_scatter(y, src3, dst3, zeros, out, src_all, dst_all, bounce_v,
                     acc, *bufs_sems):
        bufs = bufs_sems[:NBUF]
        sems = bufs_sems[NBUF:]
        c = lax.axis_index("c")
        s = lax.axis_index("s")
        row0 = s * RPT
        wid = c * NS + s
        pltpu.sync_copy(zeros.at[pl.ds(row0, RPT)], bounce_v)
        pltpu.sync_copy(bounce_v, acc.at[pl.ds(row0, RPT)])
        pltpu.sync_copy(src3.at[wid], src_all)
        pltpu.sync_copy(dst3.at[wid], dst_all)
        plsc.subcore_barrier()

        for b in range(NBUF):
            pltpu.async_copy(y.at[src_all.at[b]], bufs[b], sems[b])

        def outer(j, carry):
            k0 = j * NBUF
            for b in range(NBUF):
                k = k0 + b
                pltpu.make_async_copy(y.at[src_all.at[0]], bufs[b],
                                      sems[b]).wait()
                pltpu.sync_copy(bufs[b], acc.at[dst_all.at[k]], add=True)
                nk = k + NBUF

                @pl.when(nk < NCHUNKS)
                def _():
                    pltpu.async_copy(y.at[src_all.at[nk]], bufs[b],
                                     sems[b])

            return carry

        lax.fori_loop(0, NCHUNKS // NBUF, outer, 0)
        plsc.subcore_barrier()
        pltpu.sync_copy(acc.at[pl.ds(row0, RPT)], bounce_v)
        pltpu.sync_copy(bounce_v, out.at[pl.ds(c * NP + row0, RPT)])

    return edge_scatter


@functools.partial(
    pl.kernel,
    out_type=jax.ShapeDtypeStruct((NC * NP, DW), jnp.float32),
    mesh=_sc_mesh(),
    compiler_params=pltpu.CompilerParams(use_tc_tiling_on_sc=False),
    scratch_types=[
        pltpu.VMEM((NCHUNKS, CHUNK), jnp.int32),
        pltpu.VMEM((CHUNK, DW), jnp.float32),
        pltpu.VMEM((RPT, DW), jnp.float32),
        pltpu.VMEM_SHARED((NP, DW), jnp.float32),
        pltpu.SemaphoreType.DMA,
    ],
)
def _deg_pass(dst3, ones, zeros, out, dst_all, ones_v, bounce_v, acc, dsem):
    c = lax.axis_index("c")
    s = lax.axis_index("s")
    row0 = s * RPT
    pltpu.sync_copy(zeros.at[pl.ds(row0, RPT)], bounce_v)
    pltpu.sync_copy(bounce_v, acc.at[pl.ds(row0, RPT)])
    pltpu.sync_copy(ones, ones_v)
    wid = c * NS + s
    pltpu.sync_copy(dst3.at[wid], dst_all)
    plsc.subcore_barrier()

    def body(k, carry):
        pltpu.async_copy(ones_v, acc.at[dst_all.at[k]], dsem, add=True)
        return carry

    lax.fori_loop(0, NCHUNKS, body, 0)

    def drain(k, carry):
        pltpu.make_async_copy(ones_v, acc.at[dst_all.at[0]], dsem).wait()
        return carry

    lax.fori_loop(0, NCHUNKS, drain, 0)
    plsc.subcore_barrier()
    pltpu.sync_copy(acc.at[pl.ds(row0, RPT)], bounce_v)
    pltpu.sync_copy(bounce_v, out.at[pl.ds(c * NP + row0, RPT)])


_scat_h = _make_edge_scatter(H)

_BLK = 1000


def _row_spec(f):
    return pl.BlockSpec((_BLK, f), lambda i: (i, 0))


def _full_spec(r, f):
    return pl.BlockSpec((r, f), lambda i: (0, 0))


def _mm0_body(x_ref, w_ref, d0_ref, d1_ref, y_ref, dis_ref):
    deg = d0_ref[...] + d1_ref[...] + 1.0
    dis = lax.rsqrt(deg)
    dis_ref[...] = dis
    y_ref[...] = jnp.dot(x_ref[...], w_ref[...],
                         preferred_element_type=jnp.float32) * dis


_mm0 = pl.pallas_call(
    _mm0_body,
    grid=(N // _BLK,),
    in_specs=[
        _row_spec(D),
        _full_spec(D, H),
        _row_spec(1),
        _row_spec(1),
    ],
    out_specs=[_row_spec(H), _row_spec(1)],
    out_shape=[
        jax.ShapeDtypeStruct((N, H), jnp.float32),
        jax.ShapeDtypeStruct((N, 1), jnp.float32),
    ],
)


def _make_cmb(fin, fout, relu):
    def body(z0_ref, z1_ref, yp_ref, dis_ref, b_ref, w_ref, yo_ref):
        h = (z0_ref[...] + z1_ref[...] + yp_ref[...]) * dis_ref[...] + b_ref[...]
        if relu:
            h = jnp.maximum(h, 0.0)
        yo_ref[...] = jnp.dot(h, w_ref[...],
                              preferred_element_type=jnp.float32) * dis_ref[...]

    return pl.pallas_call(
        body,
        grid=(N // _BLK,),
        in_specs=[
            _row_spec(fin),
            _row_spec(fin),
            _row_spec(fin),
            _row_spec(1),
            _full_spec(1, fin),
            _full_spec(fin, fout),
        ],
        out_specs=_row_spec(fout),
        out_shape=jax.ShapeDtypeStruct((N, fout), jnp.float32),
    )


_cmb_enc = _make_cmb(H, H, True)


def _mid_body(z0_ref, z1_ref, yp_ref, dis_ref, b_ref, hp_ref):
    h = (z0_ref[...] + z1_ref[...] + yp_ref[...]) * dis_ref[...] + b_ref[...]
    hp_ref[...] = h * dis_ref[...]


_cmb_mid = pl.pallas_call(
    _mid_body,
    grid=(N // _BLK,),
    in_specs=[
        _row_spec(H),
        _row_spec(H),
        _row_spec(H),
        _row_spec(1),
        _full_spec(1, H),
    ],
    out_specs=_row_spec(H),
    out_shape=jax.ShapeDtypeStruct((N, H), jnp.float32),
)


def _dec_body(zh0_ref, zh1_ref, hp_ref, dis_ref, b2_ref, b4_ref, w24_ref,
              ap_ref, hs_ref):
    u = zh0_ref[...] + zh1_ref[...] + hp_ref[...]
    v = jnp.dot(u, w24_ref[...], preferred_element_type=jnp.float32)
    a = jnp.maximum(v[:, :H] * dis_ref[...] + b2_ref[...], 0.0)
    ap_ref[...] = a * dis_ref[...]
    hs_ref[...] = (v[:, H:] * dis_ref[...] + b4_ref[...]).astype(jnp.bfloat16)


_dec = pl.pallas_call(
    _dec_body,
    grid=(N // _BLK,),
    in_specs=[
        _row_spec(H),
        _row_spec(H),
        _row_spec(H),
        _row_spec(1),
        _full_spec(1, H),
        _full_spec(1, H),
        _full_spec(H, 2 * H),
    ],
    out_specs=[_row_spec(H), _row_spec(H)],
    out_shape=[
        jax.ShapeDtypeStruct((N, H), jnp.float32),
        jax.ShapeDtypeStruct((N, H), jnp.bfloat16),
    ],
)


def _fin_body(za0_ref, za1_ref, ap_ref, dis_ref, b3_ref, w3_ref, x_ref):
    u = za0_ref[...] + za1_ref[...] + ap_ref[...]
    x_ref[...] = (jnp.dot(u, w3_ref[...], preferred_element_type=jnp.float32)
                  * dis_ref[...] + b3_ref[...])


_fin = pl.pallas_call(
    _fin_body,
    grid=(N // _BLK,),
    in_specs=[
        _row_spec(H),
        _row_spec(H),
        _row_spec(H),
        _row_spec(1),
        _full_spec(1, D),
        _full_spec(H, D),
    ],
    out_specs=_row_spec(D),
    out_shape=jax.ShapeDtypeStruct((N, D), jnp.float32),
)

_BM, _BN = 1024, 512


def _gram_body(a_ref, b_ref, o_ref):
    o_ref[...] = lax.dot_general(a_ref[...], b_ref[...],
                                 (((1,), (1,)), ((), ())),
                                 preferred_element_type=jnp.float32)


_gram = pl.pallas_call(
    _gram_body,
    grid=(pl.cdiv(N, _BM), pl.cdiv(N, _BN)),
    in_specs=[
        pl.BlockSpec((_BM, H), lambda i, j: (i, 0)),
        pl.BlockSpec((_BN, H), lambda i, j: (j, 0)),
    ],
    out_specs=pl.BlockSpec((_BM, _BN), lambda i, j: (i, j)),
    out_shape=jax.ShapeDtypeStruct((N, N), jnp.float32),
)


def kernel(x, edge_index, W0, b0, W1, b1, W2, b2, W3, b3, W4, b4):
    src3 = edge_index[0].reshape(NW, NCHUNKS, CHUNK)
    dst3 = edge_index[1].reshape(NW, NCHUNKS, CHUNK)
    zeros_h = jnp.zeros((NP, H), jnp.float32)
    zeros_w = jnp.zeros((NP, DW), jnp.float32)
    ones_w = jnp.ones((CHUNK, DW), jnp.float32)

    degp = _deg_pass(dst3, ones_w, zeros_w)
    y0, dis = _mm0(x, W0, degp[:N, :1], degp[NP:NP + N, :1])

    z0 = _scat_h(y0, src3, dst3, zeros_h)
    y1 = _cmb_enc(z0[:N], z0[NP:NP + N], y0, dis, b0[None, :], W1)

    z1 = _scat_h(y1, src3, dst3, zeros_h)
    hp = _cmb_mid(z1[:N], z1[NP:NP + N], y1, dis, b1[None, :])

    zh = _scat_h(hp, src3, dst3, zeros_h)
    W24 = jnp.concatenate([W2, W4], axis=1)
    ap, hs = _dec(zh[:N], zh[NP:NP + N], hp, dis, b2[None, :], b4[None, :],
                  W24)

    za = _scat_h(ap, src3, dst3, zeros_h)
    x_ = _fin(za[:N], za[NP:NP + N], ap, dis, b3[None, :], W3)

    s_ = _gram(hs, hs)
    return (x_, s_)
